# baseline (speedup 1.0000x reference)
import functools

import jax
import jax.numpy as jnp
from jax import lax
from jax.experimental import pallas as pl
from jax.experimental.pallas import tpu as pltpu
from jax.experimental.pallas import tpu_sc as plsc

N = 10000
E = 320000
D = 128
R = 16
TPAD = 128
L = 3
NEG_SLOPE = 0.01

NC = 2
NS = 16
NW = NC * NS

BM = 1000


def _leaky(s):
    return jnp.where(s >= 0, s, NEG_SLOPE * s)


EB = 128


def _store_xw(x, wr_ref, xw_ref):
    y = jnp.dot(x, wr_ref[...], preferred_element_type=jnp.float32)
    for r in range(R):
        xw_ref[r] = y[:, r * D:(r + 1) * D]


def _mm0_body(nt_ref, emb_ref, src_ref, et_ref, wr_ref, wo_ref, b_ref,
              xw_ref, rt_ref, idx_ref):
    idx_ref[0] = et_ref[0] * N + src_ref[0]
    nt = nt_ref[0, 0, :]
    tids = lax.broadcasted_iota(jnp.int32, (BM, TPAD), 1)
    onehot = (tids == nt[:, None]).astype(jnp.float32)
    x = jnp.dot(onehot, emb_ref[...], preferred_element_type=jnp.float32)
    _store_xw(x, wr_ref, xw_ref)
    rt_ref[...] = (
        jnp.dot(x, wo_ref[...], preferred_element_type=jnp.float32) + b_ref[...]
    )


def _mm1_body(a0_ref, a1_ref, rtin_ref, wr_ref, wo_ref, b_ref, xw_ref, rt_ref):
    x = _leaky(a0_ref[...] + a1_ref[...] + rtin_ref[...])
    _store_xw(x, wr_ref, xw_ref)
    rt_ref[...] = (
        jnp.dot(x, wo_ref[...], preferred_element_type=jnp.float32) + b_ref[...]
    )


_row_spec = pl.BlockSpec((BM, D), lambda i: (i, 0))
_EBLK = E // (N // BM) // EB
_w_specs = [
    pl.BlockSpec((D, R * D), lambda i: (0, 0)),
    pl.BlockSpec((D, D), lambda i: (0, 0)),
    pl.BlockSpec((1, D), lambda i: (0, 0)),
]
_mm_out_shapes = [
    jax.ShapeDtypeStruct((R, N, D), jnp.float32),
    jax.ShapeDtypeStruct((N, D), jnp.float32),
]
_mm_out_specs = [
    pl.BlockSpec((R, BM, D), lambda i: (0, i, 0)),
    pl.BlockSpec((BM, D), lambda i: (i, 0)),
]

_mm0 = pl.pallas_call(
    _mm0_body,
    grid=(N // BM,),
    in_specs=[
        pl.BlockSpec((1, 1, BM), lambda i: (i, 0, 0)),
        pl.BlockSpec((TPAD, D), lambda i: (0, 0)),
        pl.BlockSpec((1, _EBLK, EB), lambda i: (i, 0, 0)),
        pl.BlockSpec((1, _EBLK, EB), lambda i: (i, 0, 0)),
    ] + _w_specs,
    out_specs=_mm_out_specs + [pl.BlockSpec((1, _EBLK, EB), lambda i: (i, 0, 0))],
    out_shape=_mm_out_shapes
    + [jax.ShapeDtypeStruct((N // BM, _EBLK, EB), jnp.int32)],
)

_mm1 = pl.pallas_call(
    _mm1_body,
    grid=(N // BM,),
    in_specs=[_row_spec, _row_spec, _row_spec] + _w_specs,
    out_specs=_mm_out_specs,
    out_shape=_mm_out_shapes,
)


def _fin_body(a0_ref, a1_ref, rt_ref, o_ref):
    o_ref[...] = _leaky(a0_ref[...] + a1_ref[...] + rt_ref[...])


_fin_spec = pl.BlockSpec((BM, D), lambda i: (i, 0))
_fin = pl.pallas_call(
    _fin_body,
    grid=(N // BM,),
    in_specs=[_fin_spec, _fin_spec, _fin_spec],
    out_specs=_fin_spec,
    out_shape=jax.ShapeDtypeStruct((N, D), jnp.float32),
)

CK = 80
EPW = E // NW
CPT = EPW // CK
G = 5
RPT = 632


def _edge_body(table_hbm, idx_hbm, dst_hbm, zeros_hbm, out0_hbm, out1_hbm,
               idxv, dstv, rows0, rows1, acc_sh, sem0, sem1):
    cid = lax.axis_index("c")
    sid = lax.axis_index("s")
    wid = sid * NC + cid
    ebase = pl.multiple_of(wid * EPW, 8)
    pltpu.sync_copy(idx_hbm.at[pl.ds(ebase, EPW)], idxv)
    pltpu.sync_copy(dst_hbm.at[wid], dstv)
    rbase = pl.multiple_of(jnp.minimum(sid * RPT, N - RPT), 8)
    pltpu.sync_copy(
        zeros_hbm.at[pl.ds(rbase, RPT)], acc_sh.at[pl.ds(rbase, RPT)]
    )

    rows = (rows0, rows1)
    sems = (sem0, sem1)

    def _gather(c, b):
        gsl = pl.ds(pl.multiple_of(c * CK, 8), CK)
        return pltpu.async_copy(table_hbm.at[idxv.at[gsl]], rows[b], sems[b])

    def _wait(c, b):
        gsl = pl.ds(pl.multiple_of(c * CK, 8), CK)
        pltpu.make_async_copy(
            table_hbm.at[idxv.at[gsl]], rows[b], sems[b]
        ).wait()

    plsc.subcore_barrier()

    _gather(0, 0)
    _gather(1, 1)

    def _pair(p, carry):
        for bb in range(2):
            c = 2 * p + bb
            _wait(c, bb)
            pltpu.sync_copy(rows[bb], acc_sh.at[dstv.at[c]], add=True)

            @pl.when(c + 2 < CPT)
            def _():
                _gather(c + 2, bb)

        return carry

    lax.fori_loop(0, CPT // 2, _pair, 0)
    ct = CPT - 1
    _wait(ct, 0)
    pltpu.sync_copy(rows[0], acc_sh.at[dstv.at[ct]], add=True)

    plsc.subcore_barrier()

    @pl.when(cid == 0)
    def _():
        pltpu.sync_copy(
            acc_sh.at[pl.ds(rbase, RPT)], out0_hbm.at[pl.ds(rbase, RPT)]
        )

    @pl.when(cid == 1)
    def _():
        pltpu.sync_copy(
            acc_sh.at[pl.ds(rbase, RPT)], out1_hbm.at[pl.ds(rbase, RPT)]
        )


_edge_agg = functools.partial(
    pl.kernel,
    out_type=[
        jax.ShapeDtypeStruct((N, D), jnp.float32),
        jax.ShapeDtypeStruct((N, D), jnp.float32),
    ],
    mesh=plsc.VectorSubcoreMesh(core_axis_name="c", subcore_axis_name="s"),
    scratch_types=[
        pltpu.VMEM((EPW,), jnp.int32),
        pltpu.VMEM((CPT, CK), jnp.int32),
        pltpu.VMEM((CK, D), jnp.float32),
        pltpu.VMEM((CK, D), jnp.float32),
        pltpu.VMEM_SHARED((N, D), jnp.float32),
        pltpu.SemaphoreType.DMA,
        pltpu.SemaphoreType.DMA,
    ],
)(_edge_body)


def kernel(node_type, edge_index, edge_type, node_emb, W_rel, W_root, b):
    nt = node_type.astype(jnp.int32).reshape(N // BM, 1, BM)
    src = edge_index[0].astype(jnp.int32).reshape(N // BM, _EBLK, EB)
    dst = edge_index[1].astype(jnp.int32).reshape(NW, CPT, CK)
    et = edge_type.astype(jnp.int32).reshape(N // BM, _EBLK, EB)
    emb = jnp.zeros((TPAD, D), jnp.float32).at[:node_emb.shape[0]].set(node_emb)
    zeros = jnp.zeros((N, D), jnp.float32)

    wr_all = W_rel.transpose(0, 2, 1, 3).reshape(L, D, R * D)

    rt = None
    aggs = None
    flat_idx = None
    for l in range(L):
        wr = wr_all[l]
        wo = W_root[l]
        bl = b[l].reshape(1, D)
        if l == 0:
            xw, rt, idx2 = _mm0(nt, emb, src, et, wr, wo, bl)
            flat_idx = idx2.reshape(E)
        else:
            xw, rt = _mm1(aggs[0], aggs[1], rt, wr, wo, bl)
        aggs = _edge_agg(xw.reshape(R * N, D), flat_idx, dst, zeros)

    return _fin(aggs[0], aggs[1], rt)


# speedup vs baseline: 1.0002x; 1.0002x over previous
import functools

import jax
import jax.numpy as jnp
from jax import lax
from jax.experimental import pallas as pl
from jax.experimental.pallas import tpu as pltpu
from jax.experimental.pallas import tpu_sc as plsc

N = 10000
E = 320000
D = 128
R = 16
TPAD = 128
L = 3
NEG_SLOPE = 0.01

NC = 2
NS = 16
NW = NC * NS

BM = 1000


def _leaky(s):
    return jnp.where(s >= 0, s, NEG_SLOPE * s)


EB = 128


def _store_xw(x, wr_ref, xw_ref):
    y = jnp.dot(x, wr_ref[...], preferred_element_type=jnp.float32)
    for r in range(R):
        xw_ref[r] = y[:, r * D:(r + 1) * D]


def _mm0_body(nt_ref, emb_ref, src_ref, et_ref, wr_ref, wo_ref, b_ref,
              xw_ref, rt_ref, idx_ref):
    idx_ref[0] = et_ref[0] * N + src_ref[0]
    nt = nt_ref[0, 0, :]
    tids = lax.broadcasted_iota(jnp.int32, (BM, TPAD), 1)
    onehot = (tids == nt[:, None]).astype(jnp.float32)
    x = jnp.dot(onehot, emb_ref[...], preferred_element_type=jnp.float32)
    _store_xw(x, wr_ref, xw_ref)
    rt_ref[...] = (
        jnp.dot(x, wo_ref[...], preferred_element_type=jnp.float32) + b_ref[...]
    )


def _mm1_body(a0_ref, a1_ref, rtin_ref, wr_ref, wo_ref, b_ref, xw_ref, rt_ref):
    x = _leaky(a0_ref[...] + a1_ref[...] + rtin_ref[...])
    _store_xw(x, wr_ref, xw_ref)
    rt_ref[...] = (
        jnp.dot(x, wo_ref[...], preferred_element_type=jnp.float32) + b_ref[...]
    )


_row_spec = pl.BlockSpec((BM, D), lambda i: (i, 0))
_EBLK = E // (N // BM) // EB
_w_specs = [
    pl.BlockSpec((D, R * D), lambda i: (0, 0)),
    pl.BlockSpec((D, D), lambda i: (0, 0)),
    pl.BlockSpec((1, D), lambda i: (0, 0)),
]
_mm_out_shapes = [
    jax.ShapeDtypeStruct((R, N, D), jnp.float32),
    jax.ShapeDtypeStruct((N, D), jnp.float32),
]
_mm_out_specs = [
    pl.BlockSpec((R, BM, D), lambda i: (0, i, 0)),
    pl.BlockSpec((BM, D), lambda i: (i, 0)),
]

_mm0 = pl.pallas_call(
    _mm0_body,
    grid=(N // BM,),
    in_specs=[
        pl.BlockSpec((1, 1, BM), lambda i: (i, 0, 0)),
        pl.BlockSpec((TPAD, D), lambda i: (0, 0)),
        pl.BlockSpec((1, _EBLK, EB), lambda i: (i, 0, 0)),
        pl.BlockSpec((1, _EBLK, EB), lambda i: (i, 0, 0)),
    ] + _w_specs,
    out_specs=_mm_out_specs + [pl.BlockSpec((1, _EBLK, EB), lambda i: (i, 0, 0))],
    out_shape=_mm_out_shapes
    + [jax.ShapeDtypeStruct((N // BM, _EBLK, EB), jnp.int32)],
)

_mm1 = pl.pallas_call(
    _mm1_body,
    grid=(N // BM,),
    in_specs=[_row_spec, _row_spec, _row_spec] + _w_specs,
    out_specs=_mm_out_specs,
    out_shape=_mm_out_shapes,
)


def _fin_body(a0_ref, a1_ref, rt_ref, o_ref):
    o_ref[...] = _leaky(a0_ref[...] + a1_ref[...] + rt_ref[...])


_fin_spec = pl.BlockSpec((BM, D), lambda i: (i, 0))
_fin = pl.pallas_call(
    _fin_body,
    grid=(N // BM,),
    in_specs=[_fin_spec, _fin_spec, _fin_spec],
    out_specs=_fin_spec,
    out_shape=jax.ShapeDtypeStruct((N, D), jnp.float32),
)

CK = 80
EPW = E // NW
CPT = EPW // CK
RPT = 632


def _edge_body(table_hbm, idx_hbm, dst_hbm, zeros_hbm, out0_hbm, out1_hbm,
               idxv, dstv, rows0, rows1, acc_sh, sem0, sem1):
    cid = lax.axis_index("c")
    sid = lax.axis_index("s")
    wid = sid * NC + cid
    ebase = pl.multiple_of(wid * EPW, 8)
    pltpu.sync_copy(idx_hbm.at[pl.ds(ebase, EPW)], idxv)
    pltpu.sync_copy(dst_hbm.at[wid], dstv)
    rbase = pl.multiple_of(jnp.minimum(sid * RPT, N - RPT), 8)
    pltpu.sync_copy(
        zeros_hbm.at[pl.ds(rbase, RPT)], acc_sh.at[pl.ds(rbase, RPT)]
    )

    rows = (rows0, rows1)
    sems = (sem0, sem1)

    def _gather(c, b):
        gsl = pl.ds(pl.multiple_of(c * CK, 8), CK)
        return pltpu.async_copy(table_hbm.at[idxv.at[gsl]], rows[b], sems[b])

    def _wait(c, b):
        gsl = pl.ds(pl.multiple_of(c * CK, 8), CK)
        pltpu.make_async_copy(
            table_hbm.at[idxv.at[gsl]], rows[b], sems[b]
        ).wait()

    plsc.subcore_barrier()

    _gather(0, 0)
    _gather(1, 1)

    def _pair(p, carry):
        for bb in range(2):
            c = 2 * p + bb
            _wait(c, bb)
            pltpu.sync_copy(rows[bb], acc_sh.at[dstv.at[c]], add=True)

            @pl.when(c + 2 < CPT)
            def _():
                _gather(c + 2, bb)

        return carry

    lax.fori_loop(0, CPT // 2, _pair, 0)
    ct = CPT - 1
    _wait(ct, 0)
    pltpu.sync_copy(rows[0], acc_sh.at[dstv.at[ct]], add=True)

    plsc.subcore_barrier()

    @pl.when(cid == 0)
    def _():
        pltpu.sync_copy(
            acc_sh.at[pl.ds(rbase, RPT)], out0_hbm.at[pl.ds(rbase, RPT)]
        )

    @pl.when(cid == 1)
    def _():
        pltpu.sync_copy(
            acc_sh.at[pl.ds(rbase, RPT)], out1_hbm.at[pl.ds(rbase, RPT)]
        )


_edge_agg = functools.partial(
    pl.kernel,
    out_type=[
        jax.ShapeDtypeStruct((N, D), jnp.float32),
        jax.ShapeDtypeStruct((N, D), jnp.float32),
    ],
    mesh=plsc.VectorSubcoreMesh(core_axis_name="c", subcore_axis_name="s"),
    scratch_types=[
        pltpu.VMEM((EPW,), jnp.int32),
        pltpu.VMEM((CPT, CK), jnp.int32),
        pltpu.VMEM((CK, D), jnp.float32),
        pltpu.VMEM((CK, D), jnp.float32),
        pltpu.VMEM_SHARED((N, D), jnp.float32),
        pltpu.SemaphoreType.DMA,
        pltpu.SemaphoreType.DMA,
    ],
)(_edge_body)


def kernel(node_type, edge_index, edge_type, node_emb, W_rel, W_root, b):
    nt = node_type.astype(jnp.int32).reshape(N // BM, 1, BM)
    src = edge_index[0].astype(jnp.int32).reshape(N // BM, _EBLK, EB)
    dst = edge_index[1].astype(jnp.int32).reshape(NW, CPT, CK)
    et = edge_type.astype(jnp.int32).reshape(N // BM, _EBLK, EB)
    emb = jnp.zeros((TPAD, D), jnp.float32).at[:node_emb.shape[0]].set(node_emb)
    zeros = jnp.zeros((N, D), jnp.float32)

    wr_all = W_rel.transpose(0, 2, 1, 3).reshape(L, D, R * D)

    rt = None
    aggs = None
    flat_idx = None
    for l in range(L):
        wr = wr_all[l]
        wo = W_root[l]
        bl = b[l].reshape(1, D)
        if l == 0:
            xw, rt, idx2 = _mm0(nt, emb, src, et, wr, wo, bl)
            flat_idx = idx2.reshape(E)
        else:
            xw, rt = _mm1(aggs[0], aggs[1], rt, wr, wo, bl)
        aggs = _edge_agg(xw.reshape(R * N, D), flat_idx, dst, zeros)

    return _fin(aggs[0], aggs[1], rt)
